# baseline (speedup 1.0000x reference)
import functools

import jax
import jax.numpy as jnp
from jax import lax
from jax.experimental import pallas as pl
from jax.experimental.pallas import tpu as pltpu
from jax.experimental.pallas import tpu_sc as plsc

N_NODES = 10000
N_EDGES = 320000
D = 128

NC = 2
NS = 16
NW = NC * NS
CHUNK = 128
HALVES = 2
K = 2 * HALVES * (-(-N_EDGES // (NW * CHUNK * 2 * HALVES)))
K2 = K // HALVES
E_PAD = NW * K * CHUNK
ACC_ROWS = 10112
RPS = ACC_ROWS // NS


def _sc_aggregate(x, src3, dst3, zeros):
    mesh = plsc.VectorSubcoreMesh(core_axis_name="c", subcore_axis_name="s")

    @functools.partial(
        pl.kernel,
        out_type=jax.ShapeDtypeStruct((NC * ACC_ROWS, D), jnp.float32),
        mesh=mesh,
        scratch_types=[
            pltpu.VMEM((K2, CHUNK), jnp.int32),
            pltpu.VMEM((K2, CHUNK), jnp.int32),
            pltpu.VMEM((CHUNK, D), jnp.float32),
            pltpu.VMEM((CHUNK, D), jnp.float32),
            pltpu.VMEM_SHARED((ACC_ROWS, D), jnp.float32),
            pltpu.SemaphoreType.DMA,
            pltpu.SemaphoreType.DMA,
        ],
    )
    def agg_kernel(x_hbm, src_hbm, dst_hbm, zeros_hbm, out_hbm,
                   src_v, dst_v, rows0, rows1, acc_sh, sem0, sem1):
        c = lax.axis_index("c")
        s = lax.axis_index("s")
        wid = s * NC + c

        pltpu.sync_copy(zeros_hbm, acc_sh.at[pl.ds(s * RPS, RPS)])
        plsc.subcore_barrier()

        for h in range(HALVES):
            pltpu.sync_copy(src_hbm.at[wid, pl.ds(h * K2, K2)], src_v)
            pltpu.sync_copy(dst_hbm.at[wid, pl.ds(h * K2, K2)], dst_v)
            def body(j, carry):
                pltpu.async_copy(x_hbm.at[src_v.at[j]], rows0, sem0).wait()
                pltpu.sync_copy(rows0, acc_sh.at[dst_v.at[j]], add=True)
                return carry

            lax.fori_loop(0, K2, body, 0)
        plsc.subcore_barrier()

        row0 = c * ACC_ROWS + s * RPS
        pltpu.sync_copy(acc_sh.at[pl.ds(s * RPS, RPS)], out_hbm.at[pl.ds(row0, RPS)])

    return agg_kernel(x, src3, dst3, zeros)


def _tc_layer1(aggp, x, WlT, bl, WrT):
    def body(aggp_ref, x_ref, wl_ref, bl_ref, wr_ref, out_ref):
        agg = aggp_ref[:N_NODES, :] + aggp_ref[ACC_ROWS:ACC_ROWS + N_NODES, :]
        r = (jnp.dot(agg, wl_ref[...], preferred_element_type=jnp.float32)
             + bl_ref[...]
             + jnp.dot(x_ref[...], wr_ref[...], preferred_element_type=jnp.float32))
        out_ref[...] = jnp.maximum(r, 0.0)

    return pl.pallas_call(
        body,
        out_shape=jax.ShapeDtypeStruct((N_NODES, D), jnp.float32),
    )(aggp, x, WlT, bl, WrT)


def _tc_layer2(aggp, h, WlT, bl, WrT, WlinT, blin):
    def body(aggp_ref, h_ref, wl_ref, bl_ref, wr_ref, wlin_ref, blin_ref, out_ref):
        agg = aggp_ref[:N_NODES, :] + aggp_ref[ACC_ROWS:ACC_ROWS + N_NODES, :]
        r = (jnp.dot(agg, wl_ref[...], preferred_element_type=jnp.float32)
             + bl_ref[...]
             + jnp.dot(h_ref[...], wr_ref[...], preferred_element_type=jnp.float32))
        h2 = jnp.maximum(r, 0.0)
        out_ref[...] = (jnp.dot(h2, wlin_ref[...], preferred_element_type=jnp.float32)
                        + blin_ref[...])

    return pl.pallas_call(
        body,
        out_shape=jax.ShapeDtypeStruct((N_NODES, D), jnp.float32),
    )(aggp, h, WlT, bl, WrT, WlinT, blin)


def kernel(x, edge_index, Wl1, bl1, Wr1, Wl2, bl2, Wr2, Wlin, blin):
    src = edge_index[0].astype(jnp.int32)
    dst = edge_index[1].astype(jnp.int32)
    pad = E_PAD - N_EDGES
    src3 = jnp.concatenate([src, jnp.zeros((pad,), jnp.int32)]).reshape(NW, K, CHUNK)
    dst3 = jnp.concatenate([dst, jnp.full((pad,), N_NODES, jnp.int32)]).reshape(NW, K, CHUNK)
    zeros = jnp.zeros((RPS, D), jnp.float32)

    aggp1 = _sc_aggregate(x, src3, dst3, zeros)
    h1 = _tc_layer1(aggp1, x, Wl1.T, bl1.reshape(1, D), Wr1.T)
    aggp2 = _sc_aggregate(h1, src3, dst3, zeros)
    out = _tc_layer2(aggp2, h1, Wl2.T, bl2.reshape(1, D), Wr2.T,
                     Wlin.T, blin.reshape(1, D))
    return out


# speedup vs baseline: 2.1230x; 2.1230x over previous
import functools

import jax
import jax.numpy as jnp
from jax import lax
from jax.experimental import pallas as pl
from jax.experimental.pallas import tpu as pltpu
from jax.experimental.pallas import tpu_sc as plsc

N_NODES = 10000
N_EDGES = 320000
D = 128

NC = 2
NS = 16
CHUNK = 128
KT = -(-N_EDGES // (NS * CHUNK))
KA = 102
KB = KT - KA
KM = max(KA, KB)
E_PAD = NS * KT * CHUNK
EA = NS * KA * CHUNK
ACC_ROWS = 10112
RPS = ACC_ROWS // NS


def _sc_aggregate(x, srcA, dstA, srcB, dstB, zeros):
    mesh = plsc.VectorSubcoreMesh(core_axis_name="c", subcore_axis_name="s")

    @functools.partial(
        pl.kernel,
        out_type=jax.ShapeDtypeStruct((NC * ACC_ROWS, D), jnp.float32),
        mesh=mesh,
        scratch_types=[
            pltpu.VMEM((KM, CHUNK), jnp.int32),
            pltpu.VMEM((KM, CHUNK), jnp.int32),
            pltpu.VMEM((CHUNK, D), jnp.float32),
            pltpu.VMEM_SHARED((ACC_ROWS, D), jnp.float32),
            pltpu.SemaphoreType.DMA,
        ],
    )
    def agg_kernel(x_hbm, srcA_hbm, dstA_hbm, srcB_hbm, dstB_hbm, zeros_hbm,
                   out_hbm, src_v, dst_v, rows_v, acc_sh, sem):
        c = lax.axis_index("c")
        s = lax.axis_index("s")

        @pl.when(c == 0)
        def _():
            pltpu.sync_copy(srcA_hbm.at[s], src_v.at[pl.ds(0, KA)])
            pltpu.sync_copy(dstA_hbm.at[s], dst_v.at[pl.ds(0, KA)])

        @pl.when(c == 1)
        def _():
            pltpu.sync_copy(srcB_hbm.at[s], src_v.at[pl.ds(0, KB)])
            pltpu.sync_copy(dstB_hbm.at[s], dst_v.at[pl.ds(0, KB)])

        pltpu.sync_copy(zeros_hbm, acc_sh.at[pl.ds(s * RPS, RPS)])
        plsc.subcore_barrier()

        def body(j, carry):
            pltpu.async_copy(x_hbm.at[src_v.at[j]], rows_v, sem).wait()
            pltpu.sync_copy(rows_v, acc_sh.at[dst_v.at[j]], add=True)
            return carry

        kc = lax.select(c == 0, jnp.int32(KA), jnp.int32(KB))
        lax.fori_loop(0, kc, body, 0)
        plsc.subcore_barrier()

        row0 = c * ACC_ROWS + s * RPS
        pltpu.sync_copy(acc_sh.at[pl.ds(s * RPS, RPS)], out_hbm.at[pl.ds(row0, RPS)])

    return agg_kernel(x, srcA, dstA, srcB, dstB, zeros)


def _tc_layer1(aggp, x, WlT, bl, WrT):
    def body(aggp_ref, x_ref, wl_ref, bl_ref, wr_ref, out_ref):
        agg = aggp_ref[:N_NODES, :] + aggp_ref[ACC_ROWS:ACC_ROWS + N_NODES, :]
        r = (jnp.dot(agg, wl_ref[...], preferred_element_type=jnp.float32)
             + bl_ref[...]
             + jnp.dot(x_ref[...], wr_ref[...], preferred_element_type=jnp.float32))
        out_ref[...] = jnp.maximum(r, 0.0)

    return pl.pallas_call(
        body,
        out_shape=jax.ShapeDtypeStruct((N_NODES, D), jnp.float32),
    )(aggp, x, WlT, bl, WrT)


def _tc_layer2(aggp, h, WlT, bl, WrT, WlinT, blin):
    def body(aggp_ref, h_ref, wl_ref, bl_ref, wr_ref, wlin_ref, blin_ref, out_ref):
        agg = aggp_ref[:N_NODES, :] + aggp_ref[ACC_ROWS:ACC_ROWS + N_NODES, :]
        r = (jnp.dot(agg, wl_ref[...], preferred_element_type=jnp.float32)
             + bl_ref[...]
             + jnp.dot(h_ref[...], wr_ref[...], preferred_element_type=jnp.float32))
        h2 = jnp.maximum(r, 0.0)
        out_ref[...] = (jnp.dot(h2, wlin_ref[...], preferred_element_type=jnp.float32)
                        + blin_ref[...])

    return pl.pallas_call(
        body,
        out_shape=jax.ShapeDtypeStruct((N_NODES, D), jnp.float32),
    )(aggp, h, WlT, bl, WrT, WlinT, blin)


def kernel(x, edge_index, Wl1, bl1, Wr1, Wl2, bl2, Wr2, Wlin, blin):
    src = edge_index[0].astype(jnp.int32)
    dst = edge_index[1].astype(jnp.int32)
    pad = E_PAD - N_EDGES
    src_p = jnp.concatenate([src, jnp.zeros((pad,), jnp.int32)])
    dst_p = jnp.concatenate([dst, jnp.full((pad,), N_NODES, jnp.int32)])
    srcA = src_p[:EA].reshape(NS, KA, CHUNK)
    dstA = dst_p[:EA].reshape(NS, KA, CHUNK)
    srcB = src_p[EA:].reshape(NS, KB, CHUNK)
    dstB = dst_p[EA:].reshape(NS, KB, CHUNK)
    zeros = jnp.zeros((RPS, D), jnp.float32)

    aggp1 = _sc_aggregate(x, srcA, dstA, srcB, dstB, zeros)
    h1 = _tc_layer1(aggp1, x, Wl1.T, bl1.reshape(1, D), Wr1.T)
    aggp2 = _sc_aggregate(h1, srcA, dstA, srcB, dstB, zeros)
    out = _tc_layer2(aggp2, h1, Wl2.T, bl2.reshape(1, D), Wr2.T,
                     Wlin.T, blin.reshape(1, D))
    return out
